# Initial kernel scaffold; baseline (speedup 1.0000x reference)
import jax
import jax.numpy as jnp
from jax.experimental import pallas as pl


def kernel(x, w1, b1, w2, b2, w3, b3, w4, b4, flt_cw, uidx, bpw):
    raise NotImplementedError("write your pallas kernel here")


# speedup vs baseline: 1.7330x; 1.7330x over previous
import functools

import jax
import jax.numpy as jnp
from jax import lax
from jax.experimental import pallas as pl
from jax.experimental.pallas import tpu as pltpu


def _rup(x, m):
    return ((x + m - 1) // m) * m


def _conv_pair_body(Wp, x_ref, mask_ref, w1_ref, b1_ref, w2_ref, b2_ref, o_ref):
    L = x_ref.shape[2]
    x = x_ref[0]
    cin = x.shape[0]

    def taps(img):
        for dy in (-1, 0, 1):
            for dx in (-1, 0, 1):
                sh = (-(dy * Wp + dx)) % L
                s = img if sh == 0 else pltpu.roll(img, shift=sh, axis=1)
                yield (dy + 1) * 3 + (dx + 1), s

    if cin == 1:
        stk = jnp.concatenate([s for _, s in taps(x)], axis=0)
        h = jnp.dot(w1_ref[...], stk, preferred_element_type=jnp.float32)
    else:
        h = None
        for t, s in taps(x):
            c = jnp.dot(w1_ref[t], s, preferred_element_type=jnp.float32)
            h = c if h is None else h + c
    h = (h + b1_ref[...]) * mask_ref[...]

    o = None
    for t, s in taps(h):
        c = jnp.dot(w2_ref[t], s, preferred_element_type=jnp.float32)
        o = c if o is None else o + c
    o_ref[0] = o + b2_ref[...]


def _tap_mats(w, b):
    co, ci = w.shape[0], w.shape[1]
    wt = jnp.transpose(w, (2, 3, 0, 1)).reshape(9, co, ci).astype(jnp.float32)
    return wt, b.reshape(co, 1).astype(jnp.float32)


def _conv_pair(x, w1, b1, w2, b2):
    N, Cin, H, W = x.shape
    Cmid, Cout = w1.shape[0], w2.shape[0]
    Wp = _rup(W + 2, 128)
    Hp = H + 2
    L = Hp * Wp

    xp = jnp.pad(x, ((0, 0), (0, 0), (1, 1), (0, Wp - W))).reshape(N, Cin, L)
    mask = jnp.pad(jnp.ones((H, W), jnp.float32), ((1, 1), (0, Wp - W))).reshape(1, L)

    if Cin == 1:
        w1m = w1.reshape(Cmid, 9).astype(jnp.float32)
        w1_spec = pl.BlockSpec((Cmid, 9), lambda n: (0, 0))
    else:
        w1m, _ = _tap_mats(w1, b1)
        w1_spec = pl.BlockSpec((9, Cmid, Cin), lambda n: (0, 0, 0))
    b1m = b1.reshape(Cmid, 1).astype(jnp.float32)
    w2m, b2m = _tap_mats(w2, b2)

    out = pl.pallas_call(
        functools.partial(_conv_pair_body, Wp),
        out_shape=jax.ShapeDtypeStruct((N, Cout, L), jnp.float32),
        grid=(N,),
        in_specs=[
            pl.BlockSpec((1, Cin, L), lambda n: (n, 0, 0)),
            pl.BlockSpec((1, L), lambda n: (0, 0)),
            w1_spec,
            pl.BlockSpec((Cmid, 1), lambda n: (0, 0)),
            pl.BlockSpec((9, Cout, Cmid), lambda n: (0, 0, 0)),
            pl.BlockSpec((Cout, 1), lambda n: (0, 0)),
        ],
        out_specs=pl.BlockSpec((1, Cout, L), lambda n: (n, 0, 0)),
        compiler_params=pltpu.CompilerParams(
            dimension_semantics=("parallel",),
            vmem_limit_bytes=64 * 1024 * 1024),
    )(xp, mask, w1m, b1m, w2m, b2m)

    return out.reshape(N, Cout, Hp, Wp)[:, :, 1:H + 1, :W]


def _filter_body(x_ref, flt_ref, o_ref):
    o_ref[...] = jnp.dot(x_ref[...], flt_ref[...],
                         preferred_element_type=jnp.float32,
                         precision=lax.Precision.HIGHEST).astype(jnp.bfloat16)


def _fbp_filter(rows, flt):
    R, det = rows.shape
    tr = 576 if R % 576 == 0 else max(t for t in (512, 256, 128, 64, 32, 16, 8)
                                      if R % t == 0)
    return pl.pallas_call(
        _filter_body,
        out_shape=jax.ShapeDtypeStruct((R, det), jnp.bfloat16),
        grid=(R // tr,),
        in_specs=[pl.BlockSpec((tr, det), lambda r: (r, 0)),
                  pl.BlockSpec((det, det), lambda r: (0, 0))],
        out_specs=pl.BlockSpec((tr, det), lambda r: (r, 0)),
        compiler_params=pltpu.CompilerParams(dimension_semantics=("parallel",)),
    )(rows, flt)


def _bp_body(vb, det, f_ref, i0_ref, wa_ref, wb_ref, o_ref):
    j = pl.program_id(1)
    tn = i0_ref.shape[1]

    kk = lax.broadcasted_iota(jnp.int32, (1, det, 1), 1).astype(jnp.bfloat16)
    d = i0_ref[...][:, None, :] - kk
    zero = jnp.zeros((), jnp.bfloat16)
    hat = jnp.where(d == 0, wa_ref[...][:, None, :],
                    jnp.where(d == -1, wb_ref[...][:, None, :], zero))

    K = vb * det
    off = pl.multiple_of(j * K, K)
    fblk = f_ref[:, pl.ds(off, K)]
    contrib = jnp.dot(fblk, hat.reshape(K, tn),
                      preferred_element_type=jnp.float32)

    @pl.when(j == 0)
    def _():
        o_ref[...] = contrib

    @pl.when(j > 0)
    def _():
        o_ref[...] += contrib


def _backproject(filt, uidx, bpw, img_dim):
    B, V, det = filt.shape
    npix = img_dim * img_dim
    vb = 16
    VP = _rup(V, vb)
    tn = 256

    i0 = jnp.floor(uidx)
    fr = uidx - i0
    pv = VP - V
    i0b = jnp.pad(i0, ((0, pv), (0, 0))).astype(jnp.bfloat16)
    wab = jnp.pad(bpw * (1.0 - fr), ((0, pv), (0, 0))).astype(jnp.bfloat16)
    wbb = jnp.pad(bpw * fr, ((0, pv), (0, 0))).astype(jnp.bfloat16)

    f2d = jnp.pad(filt, ((0, 0), (0, pv), (0, 0))).reshape(B, VP * det)

    return pl.pallas_call(
        functools.partial(_bp_body, vb, det),
        out_shape=jax.ShapeDtypeStruct((B, npix), jnp.float32),
        grid=(npix // tn, VP // vb),
        in_specs=[
            pl.BlockSpec((B, VP * det), lambda t, j: (0, 0)),
            pl.BlockSpec((vb, tn), lambda t, j: (j, t)),
            pl.BlockSpec((vb, tn), lambda t, j: (j, t)),
            pl.BlockSpec((vb, tn), lambda t, j: (j, t)),
        ],
        out_specs=pl.BlockSpec((B, tn), lambda t, j: (0, t)),
        compiler_params=pltpu.CompilerParams(
            dimension_semantics=("parallel", "arbitrary"),
            vmem_limit_bytes=48 * 1024 * 1024),
    )(f2d, i0b, wab, wbb)


def kernel(x, w1, b1, w2, b2, w3, b3, w4, b4, flt_cw, uidx, bpw):
    img_dim = 128
    h = _conv_pair(x, w1, b1, w2, b2)
    N, C, V, D = h.shape

    filt = _fbp_filter(h.reshape(N * C * V, D), flt_cw)
    img = _backproject(filt.reshape(N * C, V, D), uidx, bpw, img_dim)

    out = _conv_pair(img.reshape(N, C, img_dim, img_dim), w3, b3, w4, b4)
    return out


# speedup vs baseline: 2.1252x; 1.2263x over previous
import functools

import jax
import jax.numpy as jnp
from jax import lax
from jax.experimental import pallas as pl
from jax.experimental.pallas import tpu as pltpu


def _rup(x, m):
    return ((x + m - 1) // m) * m


def _conv_pair_body(Wp, x_ref, mask_ref, w1_ref, b1_ref, w2_ref, b2_ref, o_ref):
    L = x_ref.shape[2]
    x = x_ref[0]
    cin = x.shape[0]

    def taps(img):
        for dy in (-1, 0, 1):
            for dx in (-1, 0, 1):
                sh = (-(dy * Wp + dx)) % L
                s = img if sh == 0 else pltpu.roll(img, shift=sh, axis=1)
                yield (dy + 1) * 3 + (dx + 1), s

    if cin == 1:
        stk = jnp.concatenate([s for _, s in taps(x)], axis=0)
        h = jnp.dot(w1_ref[...], stk, preferred_element_type=jnp.float32)
    else:
        h = None
        for t, s in taps(x):
            c = jnp.dot(w1_ref[t], s, preferred_element_type=jnp.float32)
            h = c if h is None else h + c
    h = (h + b1_ref[...]) * mask_ref[...]

    o = None
    for t, s in taps(h):
        c = jnp.dot(w2_ref[t], s, preferred_element_type=jnp.float32)
        o = c if o is None else o + c
    o_ref[0] = o + b2_ref[...]


def _tap_mats(w, b):
    co, ci = w.shape[0], w.shape[1]
    wt = jnp.transpose(w, (2, 3, 0, 1)).reshape(9, co, ci).astype(jnp.float32)
    return wt, b.reshape(co, 1).astype(jnp.float32)


def _conv_pair(x, w1, b1, w2, b2):
    N, Cin, H, W = x.shape
    Cmid, Cout = w1.shape[0], w2.shape[0]
    Wp = _rup(W + 2, 128)
    Hp = H + 2
    L = Hp * Wp

    xp = jnp.pad(x, ((0, 0), (0, 0), (1, 1), (0, Wp - W))).reshape(N, Cin, L)
    mask = jnp.pad(jnp.ones((H, W), jnp.float32), ((1, 1), (0, Wp - W))).reshape(1, L)

    if Cin == 1:
        w1m = w1.reshape(Cmid, 9).astype(jnp.float32)
        w1_spec = pl.BlockSpec((Cmid, 9), lambda n: (0, 0))
    else:
        w1m, _ = _tap_mats(w1, b1)
        w1_spec = pl.BlockSpec((9, Cmid, Cin), lambda n: (0, 0, 0))
    b1m = b1.reshape(Cmid, 1).astype(jnp.float32)
    w2m, b2m = _tap_mats(w2, b2)

    out = pl.pallas_call(
        functools.partial(_conv_pair_body, Wp),
        out_shape=jax.ShapeDtypeStruct((N, Cout, L), jnp.float32),
        grid=(N,),
        in_specs=[
            pl.BlockSpec((1, Cin, L), lambda n: (n, 0, 0)),
            pl.BlockSpec((1, L), lambda n: (0, 0)),
            w1_spec,
            pl.BlockSpec((Cmid, 1), lambda n: (0, 0)),
            pl.BlockSpec((9, Cout, Cmid), lambda n: (0, 0, 0)),
            pl.BlockSpec((Cout, 1), lambda n: (0, 0)),
        ],
        out_specs=pl.BlockSpec((1, Cout, L), lambda n: (n, 0, 0)),
        compiler_params=pltpu.CompilerParams(
            dimension_semantics=("parallel",),
            vmem_limit_bytes=64 * 1024 * 1024),
    )(xp, mask, w1m, b1m, w2m, b2m)

    return out.reshape(N, Cout, Hp, Wp)[:, :, 1:H + 1, :W]


def _filter_body(x_ref, flt_ref, o_ref):
    o_ref[...] = jnp.dot(x_ref[...], flt_ref[...],
                         preferred_element_type=jnp.float32,
                         precision=lax.Precision.HIGHEST).astype(jnp.bfloat16)


def _fbp_filter(rows, flt):
    R, det = rows.shape
    tr = 576 if R % 576 == 0 else max(t for t in (512, 256, 128, 64, 32, 16, 8)
                                      if R % t == 0)
    return pl.pallas_call(
        _filter_body,
        out_shape=jax.ShapeDtypeStruct((R, det), jnp.bfloat16),
        grid=(R // tr,),
        in_specs=[pl.BlockSpec((tr, det), lambda r: (r, 0)),
                  pl.BlockSpec((det, det), lambda r: (0, 0))],
        out_specs=pl.BlockSpec((tr, det), lambda r: (r, 0)),
        compiler_params=pltpu.CompilerParams(dimension_semantics=("parallel",)),
    )(rows, flt)


def _bp_body(vb, det, nj, f_ref, i0_ref, wa_ref, wb_ref, o_ref):
    tn = i0_ref.shape[2]
    K = vb * det
    kk = lax.broadcasted_iota(jnp.int32, (1, det, 1), 1).astype(jnp.bfloat16)
    zero = jnp.zeros((), jnp.bfloat16)

    acc = None
    for j in range(nj):
        i0 = i0_ref[pl.ds(j * vb, vb)]
        d = i0 - kk
        hat = jnp.where(d == 0, wa_ref[pl.ds(j * vb, vb)],
                        jnp.where(d == -1, wb_ref[pl.ds(j * vb, vb)], zero))
        fblk = f_ref[:, pl.ds(j * K, K)]
        c = jnp.dot(fblk, hat.reshape(K, tn),
                    preferred_element_type=jnp.float32)
        acc = c if acc is None else acc + c
    o_ref[...] = acc


def _backproject(filt, uidx, bpw, img_dim):
    B, V, det = filt.shape
    npix = img_dim * img_dim
    vb = 16
    VP = _rup(V, vb)
    tn = 256

    i0 = jnp.floor(uidx)
    fr = uidx - i0
    pv = VP - V

    def prep(a):
        return jnp.pad(a, ((0, pv), (0, 0))).astype(jnp.bfloat16).reshape(VP, 1, npix)

    i0b, wab, wbb = prep(i0), prep(bpw * (1.0 - fr)), prep(bpw * fr)
    f2d = jnp.pad(filt, ((0, 0), (0, pv), (0, 0))).reshape(B, VP * det)

    return pl.pallas_call(
        functools.partial(_bp_body, vb, det, VP // vb),
        out_shape=jax.ShapeDtypeStruct((B, npix), jnp.float32),
        grid=(npix // tn,),
        in_specs=[
            pl.BlockSpec((B, VP * det), lambda t: (0, 0)),
            pl.BlockSpec((VP, 1, tn), lambda t: (0, 0, t)),
            pl.BlockSpec((VP, 1, tn), lambda t: (0, 0, t)),
            pl.BlockSpec((VP, 1, tn), lambda t: (0, 0, t)),
        ],
        out_specs=pl.BlockSpec((B, tn), lambda t: (0, t)),
        compiler_params=pltpu.CompilerParams(
            dimension_semantics=("parallel",),
            vmem_limit_bytes=48 * 1024 * 1024),
    )(f2d, i0b, wab, wbb)


def kernel(x, w1, b1, w2, b2, w3, b3, w4, b4, flt_cw, uidx, bpw):
    img_dim = 128
    h = _conv_pair(x, w1, b1, w2, b2)
    N, C, V, D = h.shape

    filt = _fbp_filter(h.reshape(N * C * V, D), flt_cw)
    img = _backproject(filt.reshape(N * C, V, D), uidx, bpw, img_dim)

    out = _conv_pair(img.reshape(N, C, img_dim, img_dim), w3, b3, w4, b4)
    return out


# speedup vs baseline: 2.3234x; 1.0933x over previous
import functools

import jax
import jax.numpy as jnp
from jax import lax
from jax.experimental import pallas as pl
from jax.experimental.pallas import tpu as pltpu


def _rup(x, m):
    return ((x + m - 1) // m) * m


def _conv_pair_body(Wp, x_ref, mask_ref, w1_ref, b1_ref, w2_ref, b2_ref, o_ref):
    L = x_ref.shape[2]
    x = x_ref[0]
    cin = x.shape[0]

    def taps(img):
        for dy in (-1, 0, 1):
            for dx in (-1, 0, 1):
                sh = (-(dy * Wp + dx)) % L
                s = img if sh == 0 else pltpu.roll(img, shift=sh, axis=1)
                yield (dy + 1) * 3 + (dx + 1), s

    if cin == 1:
        stk = jnp.concatenate([s for _, s in taps(x)], axis=0)
        h = jnp.dot(w1_ref[...], stk, preferred_element_type=jnp.float32)
    else:
        h = None
        for t, s in taps(x):
            c = jnp.dot(w1_ref[t], s, preferred_element_type=jnp.float32)
            h = c if h is None else h + c
    h = (h + b1_ref[...]) * mask_ref[...]

    o = None
    for t, s in taps(h):
        c = jnp.dot(w2_ref[t], s, preferred_element_type=jnp.float32)
        o = c if o is None else o + c
    o_ref[0] = o + b2_ref[...]


def _tap_mats(w, b):
    co, ci = w.shape[0], w.shape[1]
    wt = jnp.transpose(w, (2, 3, 0, 1)).reshape(9, co, ci).astype(jnp.float32)
    return wt, b.reshape(co, 1).astype(jnp.float32)


def _conv_pair(x, w1, b1, w2, b2):
    N, Cin, H, W = x.shape
    Cmid, Cout = w1.shape[0], w2.shape[0]
    Wp = _rup(W + 2, 128)
    Hp = H + 2
    L = Hp * Wp

    xp = jnp.pad(x, ((0, 0), (0, 0), (1, 1), (0, Wp - W))).reshape(N, Cin, L)
    mask = jnp.pad(jnp.ones((H, W), jnp.float32), ((1, 1), (0, Wp - W))).reshape(1, L)

    if Cin == 1:
        w1m = w1.reshape(Cmid, 9).astype(jnp.float32)
        w1_spec = pl.BlockSpec((Cmid, 9), lambda n: (0, 0))
    else:
        w1m, _ = _tap_mats(w1, b1)
        w1_spec = pl.BlockSpec((9, Cmid, Cin), lambda n: (0, 0, 0))
    b1m = b1.reshape(Cmid, 1).astype(jnp.float32)
    w2m, b2m = _tap_mats(w2, b2)

    out = pl.pallas_call(
        functools.partial(_conv_pair_body, Wp),
        out_shape=jax.ShapeDtypeStruct((N, Cout, L), jnp.float32),
        grid=(N,),
        in_specs=[
            pl.BlockSpec((1, Cin, L), lambda n: (n, 0, 0)),
            pl.BlockSpec((1, L), lambda n: (0, 0)),
            w1_spec,
            pl.BlockSpec((Cmid, 1), lambda n: (0, 0)),
            pl.BlockSpec((9, Cout, Cmid), lambda n: (0, 0, 0)),
            pl.BlockSpec((Cout, 1), lambda n: (0, 0)),
        ],
        out_specs=pl.BlockSpec((1, Cout, L), lambda n: (n, 0, 0)),
        compiler_params=pltpu.CompilerParams(
            dimension_semantics=("parallel",),
            vmem_limit_bytes=64 * 1024 * 1024),
    )(xp, mask, w1m, b1m, w2m, b2m)

    return out.reshape(N, Cout, Hp, Wp)[:, :, 1:H + 1, :W]


def _filter_body(x_ref, flt_ref, o_ref):
    o_ref[...] = jnp.dot(x_ref[...], flt_ref[...],
                         preferred_element_type=jnp.float32).astype(jnp.bfloat16)


def _fbp_filter(rows, flt):
    R, det = rows.shape
    tr = 576 if R % 576 == 0 else max(t for t in (512, 256, 128, 64, 32, 16, 8)
                                      if R % t == 0)
    return pl.pallas_call(
        _filter_body,
        out_shape=jax.ShapeDtypeStruct((R, det), jnp.bfloat16),
        grid=(R // tr,),
        in_specs=[pl.BlockSpec((tr, det), lambda r: (r, 0)),
                  pl.BlockSpec((det, det), lambda r: (0, 0))],
        out_specs=pl.BlockSpec((tr, det), lambda r: (r, 0)),
        compiler_params=pltpu.CompilerParams(dimension_semantics=("parallel",)),
    )(rows, flt)


def _bp_body(vb, det, nj, f_ref, i0_ref, wa_ref, wb_ref, o_ref):
    tn = i0_ref.shape[2]
    K = vb * det
    kk = lax.broadcasted_iota(jnp.int32, (1, det, 1), 1).astype(jnp.bfloat16)
    kkm1 = kk - jnp.bfloat16(1.0)
    zero = jnp.zeros((), jnp.bfloat16)

    acc = None
    for j in range(nj):
        i0 = i0_ref[pl.ds(j * vb, vb)]
        hat = jnp.where(i0 == kk, wa_ref[pl.ds(j * vb, vb)],
                        jnp.where(i0 == kkm1, wb_ref[pl.ds(j * vb, vb)], zero))
        fblk = f_ref[:, pl.ds(j * K, K)]
        c = jnp.dot(fblk, hat.reshape(K, tn),
                    preferred_element_type=jnp.float32)
        acc = c if acc is None else acc + c
    o_ref[...] = acc


def _backproject(filt, uidx, bpw, img_dim):
    B, V, det = filt.shape
    npix = img_dim * img_dim
    vb = 16
    VP = _rup(V, vb)
    tn = 256

    i0 = jnp.floor(uidx)
    fr = uidx - i0
    pv = VP - V

    def prep(a):
        return jnp.pad(a, ((0, pv), (0, 0))).astype(jnp.bfloat16).reshape(VP, 1, npix)

    i0b, wab, wbb = prep(i0), prep(bpw * (1.0 - fr)), prep(bpw * fr)
    f2d = jnp.pad(filt, ((0, 0), (0, pv), (0, 0))).reshape(B, VP * det)

    return pl.pallas_call(
        functools.partial(_bp_body, vb, det, VP // vb),
        out_shape=jax.ShapeDtypeStruct((B, npix), jnp.float32),
        grid=(npix // tn,),
        in_specs=[
            pl.BlockSpec((B, VP * det), lambda t: (0, 0)),
            pl.BlockSpec((VP, 1, tn), lambda t: (0, 0, t)),
            pl.BlockSpec((VP, 1, tn), lambda t: (0, 0, t)),
            pl.BlockSpec((VP, 1, tn), lambda t: (0, 0, t)),
        ],
        out_specs=pl.BlockSpec((B, tn), lambda t: (0, t)),
        compiler_params=pltpu.CompilerParams(
            dimension_semantics=("parallel",),
            vmem_limit_bytes=48 * 1024 * 1024),
    )(f2d, i0b, wab, wbb)


def kernel(x, w1, b1, w2, b2, w3, b3, w4, b4, flt_cw, uidx, bpw):
    img_dim = 128
    h = _conv_pair(x, w1, b1, w2, b2)
    N, C, V, D = h.shape

    filt = _fbp_filter(h.reshape(N * C * V, D), flt_cw)
    img = _backproject(filt.reshape(N * C, V, D), uidx, bpw, img_dim)

    out = _conv_pair(img.reshape(N, C, img_dim, img_dim), w3, b3, w4, b4)
    return out
